# Initial kernel scaffold; baseline (speedup 1.0000x reference)
import jax
import jax.numpy as jnp
from jax.experimental import pallas as pl


def kernel(graph_embed, utterance_embed, mention_index, mention_batch_index, sel_indices, sel_batch_indices, sel_group_indices, grp_batch_indices, last_indices, intent_indices, att_W, att_V, layer_w, layer_b):
    raise NotImplementedError("write your pallas kernel here")


# speedup vs baseline: 3.6269x; 3.6269x over previous
import functools

import jax
import jax.numpy as jnp
from jax import lax
from jax.experimental import pallas as pl
from jax.experimental.pallas import tpu as pltpu
from jax.experimental.pallas import tpu_sc as plsc

NC, NS = 2, 16
NW = NC * NS
L = 16

NV = 100000
D = 64
NM = 16384
NSEL = 32768
B = 16
QROWS = 272

M_PER_W = NM // NW
S_PER_W = NSEL // NW
DMA_CHUNK = 128

_mesh = plsc.VectorSubcoreMesh(
    core_axis_name="c", subcore_axis_name="s", num_cores=NC, num_subcores=NS)
_sc_params = pltpu.CompilerParams(use_tc_tiling_on_sc=False,
                                  needs_layout_passes=False)

_f32 = jnp.float32
_i32 = jnp.int32


@functools.partial(
    pl.kernel,
    out_type=[jax.ShapeDtypeStruct((NM, D), _f32),
              jax.ShapeDtypeStruct((2 * B, D), _f32)],
    mesh=_mesh,
    compiler_params=_sc_params,
    scratch_types=[
        pltpu.VMEM((M_PER_W,), _i32),
        pltpu.VMEM((M_PER_W, D), _f32),
        pltpu.VMEM((2 * B,), _i32),
        pltpu.VMEM((2 * B, D), _f32),
        pltpu.SemaphoreType.DMA,
    ],
)
def _gather_stage(graph_hbm, mention_idx_hbm, last_hbm,
                  me_out, sp_out, idx_v, rows_v, last_v, sp_v, sem):
    wid = lax.axis_index("s") * NC + lax.axis_index("c")
    base = wid * M_PER_W
    pltpu.sync_copy(mention_idx_hbm.at[pl.ds(base, M_PER_W)], idx_v)
    cps = [
        pltpu.async_copy(
            graph_hbm.at[idx_v.at[pl.ds(j * DMA_CHUNK, DMA_CHUNK)]],
            rows_v.at[pl.ds(j * DMA_CHUNK, DMA_CHUNK)],
            sem)
        for j in range(M_PER_W // DMA_CHUNK)
    ]
    for cp in cps:
        cp.wait()
    pltpu.sync_copy(rows_v, me_out.at[pl.ds(base, M_PER_W)])

    @pl.when(wid == 0)
    def _():
        pltpu.sync_copy(last_hbm, last_v)
        for r in range(2):
            v = last_v[pl.ds(r * L, L)]
            last_v[pl.ds(r * L, L)] = jnp.minimum(v, NV - 1)
        pltpu.async_copy(graph_hbm.at[last_v], sp_v, sem).wait()
        pltpu.sync_copy(sp_v, sp_out)


def _dense_body(me_ref, mbi_ref, utt_ref, attw_ref, attv_ref, sp_ref,
                last_ref, grp_ref, intent_ref, lw_ref, lb_ref, q_ref):
    hi = jax.lax.Precision.HIGHEST
    me = me_ref[...]
    h = jnp.tanh(lax.dot_general(me, attw_ref[...], (((1,), (0,)), ((), ())),
                                 preferred_element_type=_f32))
    beta = jnp.sum(h * attv_ref[...][None, :], axis=1, keepdims=True)
    eb = jnp.exp(beta)
    iota_b = lax.broadcasted_iota(_i32, (B, NM), 0)
    onehot = (iota_b == mbi_ref[...].reshape(1, NM)).astype(_f32)
    denom = lax.dot_general(onehot, eb, (((1,), (0,)), ((), ())),
                            preferred_element_type=_f32)
    num = lax.dot_general(onehot, me * eb, (((1,), (0,)), ((), ())),
                          preferred_element_type=_f32)
    portrait = num / jnp.where(denom > 0.0, denom, 1.0)
    utt = utt_ref[...]
    context = jnp.concatenate([utt, portrait], axis=1)

    iota16 = lax.broadcasted_iota(_i32, (B, B), 1)
    r_row = lax.broadcasted_iota(_i32, (B * B, B), 0)
    r_col = lax.broadcasted_iota(_i32, (B * B, B), 1)
    selb_oh = (r_row // B == r_col).astype(_f32)
    selg_oh = (r_row % B == r_col).astype(_f32)
    u_rep = lax.dot_general(selb_oh, utt, (((1,), (0,)), ((), ())),
                            precision=hi, preferred_element_type=_f32)
    p_rep = lax.dot_general(selb_oh, portrait, (((1,), (0,)), ((), ())),
                            precision=hi, preferred_element_type=_f32)

    for i in range(2):
        grp = grp_ref[i]
        g_oh = (grp[:, None] == iota16).astype(_f32)
        tiled_ctx = lax.dot_general(g_oh, context, (((1,), (0,)), ((), ())),
                                    precision=hi, preferred_element_type=_f32)
        lr = last_ref[i]
        spm = sp_ref[pl.ds(i * B, B), :] * (lr < NV).astype(_f32)[:, None]
        gctx = jnp.concatenate([tiled_ctx, spm], axis=1)
        logits = lax.dot_general(gctx, lw_ref[i], (((1,), (1,)), ((), ())),
                                 precision=hi, preferred_element_type=_f32)
        wmat = jax.nn.sigmoid(logits + lb_ref[i][None, :])
        ii = intent_ref[i][:, None]
        w = jnp.where(ii == 0, wmat[:, 0:1],
                      jnp.where(ii == 1, wmat[:, 1:2], wmat[:, 2:3]))
        w_rep = lax.dot_general(selg_oh, w, (((1,), (0,)), ((), ())),
                                precision=hi, preferred_element_type=_f32)
        qi = u_rep * w_rep + p_rep * (1.0 - w_rep)
        q_ref[i, pl.ds(0, B * B), :] = qi
        q_ref[i, pl.ds(B * B, QROWS - B * B), :] = jnp.zeros(
            (QROWS - B * B, D), _f32)


_dense_call = pl.pallas_call(
    _dense_body,
    out_shape=jax.ShapeDtypeStruct((2, QROWS, D), _f32),
)


@functools.partial(
    pl.kernel,
    out_type=jax.ShapeDtypeStruct((2, NSEL), _f32),
    mesh=_mesh,
    compiler_params=_sc_params,
    scratch_types=[
        pltpu.VMEM((2, QROWS, D), _f32),
        pltpu.VMEM((S_PER_W,), _i32),
        pltpu.VMEM((S_PER_W,), _i32),
        pltpu.VMEM((S_PER_W,), _i32),
        pltpu.VMEM((S_PER_W,), _i32),
        pltpu.VMEM((S_PER_W, D), _f32),
        pltpu.VMEM((S_PER_W,), _f32),
        pltpu.SemaphoreType.DMA,
    ],
)
def _walk_stage(graph_hbm, q_hbm, sel_hbm, selb_hbm, selg_hbm,
                out_hbm, q_v, sel_v, b_v, g_v, idxc_v, rows_v, out_v, sem):
    wid = lax.axis_index("s") * NC + lax.axis_index("c")
    base = wid * S_PER_W
    pltpu.sync_copy(q_hbm, q_v)
    lanes = lax.iota(_i32, L)
    for i in range(2):
        pltpu.sync_copy(sel_hbm.at[i, pl.ds(base, S_PER_W)], sel_v)
        pltpu.sync_copy(selb_hbm.at[i, pl.ds(base, S_PER_W)], b_v)
        pltpu.sync_copy(selg_hbm.at[i, pl.ds(base, S_PER_W)], g_v)

        def clamp_body(t, carry):
            off = t * L
            idxc_v[pl.ds(off, L)] = jnp.minimum(sel_v[pl.ds(off, L)], NV - 1)
            return carry
        lax.fori_loop(0, S_PER_W // L, clamp_body, 0)

        cps = [
            pltpu.async_copy(
                graph_hbm.at[idxc_v.at[pl.ds(j * DMA_CHUNK, DMA_CHUNK)]],
                rows_v.at[pl.ds(j * DMA_CHUNK, DMA_CHUNK)],
                sem)
            for j in range(S_PER_W // DMA_CHUNK)
        ]
        for cp in cps:
            cp.wait()

        ivec = jnp.full((L,), i, _i32)

        def group_body(g, carry):
            off = g * L
            sv = sel_v[pl.ds(off, L)]
            c16 = b_v[pl.ds(off, L)] * B + g_v[pl.ds(off, L)]
            c16 = jnp.where(sv >= NV, B * B, c16)
            row16 = lanes + off
            accs = [jnp.zeros((L,), _f32) for _ in range(4)]
            for d in range(D):
                dvec = jnp.full((L,), d, _i32)
                qd = plsc.load_gather(q_v, [ivec, c16, dvec])
                rd = plsc.load_gather(rows_v, [row16, dvec])
                accs[d % 4] = accs[d % 4] + qd * rd
            out_v[pl.ds(off, L)] = (accs[0] + accs[1]) + (accs[2] + accs[3])
            return carry
        lax.fori_loop(0, S_PER_W // L, group_body, 0)

        pltpu.sync_copy(out_v, out_hbm.at[i, pl.ds(base, S_PER_W)])


def kernel(graph_embed, utterance_embed, mention_index, mention_batch_index,
           sel_indices, sel_batch_indices, sel_group_indices,
           grp_batch_indices, last_indices, intent_indices,
           att_W, att_V, layer_w, layer_b):
    mi = mention_index.astype(_i32)
    mbi = mention_batch_index.astype(_i32)
    sel = sel_indices.astype(_i32)
    selb = sel_batch_indices.astype(_i32)
    selg = sel_group_indices.astype(_i32)
    grp = grp_batch_indices.astype(_i32)
    last = last_indices.astype(_i32)
    intent = intent_indices.astype(_i32)

    me, sp = _gather_stage(graph_embed, mi, last.reshape(2 * B))
    q = _dense_call(me, mbi, utterance_embed, att_W, att_V, sp,
                    last, grp, intent, layer_w, layer_b)
    return _walk_stage(graph_embed, q, sel, selb, selg)
